# baseline (device time: 31478 ns/iter reference)
import jax
import jax.numpy as jnp
from jax import lax
from jax.experimental import pallas as pl
from jax.experimental.pallas import tpu as pltpu

N_DEV = 32


def kernel(dy, W):
    m, k = dy.shape
    n = W.shape[0]
    chunk = m // N_DEV

    def body(dy_ref, w_ref, out_ref, pbuf, rs_buf, g_buf,
             send1, recv1, send2, recv2):
        my = lax.axis_index("i")

        partial = lax.dot_general(
            dy_ref[...].astype(jnp.bfloat16),
            w_ref[...].astype(jnp.bfloat16),
            (((1,), (1,)), ((), ())),
            preferred_element_type=jnp.float32,
        )
        pbuf[...] = partial.astype(jnp.bfloat16).reshape(N_DEV, chunk, n)

        barrier = pltpu.get_barrier_semaphore()
        for s in range(1, N_DEV):
            peer = lax.rem(my + s, N_DEV)
            pl.semaphore_signal(
                barrier, inc=1, device_id=(peer,),
                device_id_type=pl.DeviceIdType.MESH,
            )
        pl.semaphore_wait(barrier, N_DEV - 1)

        p1 = []
        for s in range(1, N_DEV):
            dst = lax.rem(my + s, N_DEV)
            rdma = pltpu.make_async_remote_copy(
                src_ref=pbuf.at[dst],
                dst_ref=rs_buf.at[my],
                send_sem=send1.at[s],
                recv_sem=recv1.at[my],
                device_id=(dst,),
                device_id_type=pl.DeviceIdType.MESH,
            )
            rdma.start()
            p1.append(rdma)

        rs_buf[my, :, :] = pbuf[my]

        for s in range(1, N_DEV):
            src = lax.rem(my - s + N_DEV, N_DEV)
            pltpu.make_async_remote_copy(
                src_ref=pbuf.at[src],
                dst_ref=rs_buf.at[src],
                send_sem=send1.at[s],
                recv_sem=recv1.at[src],
                device_id=(src,),
                device_id_type=pl.DeviceIdType.MESH,
            ).wait_recv()

        acc = jnp.sum(rs_buf[...].astype(jnp.float32), axis=0)
        g_buf[my, :, :] = acc.astype(jnp.bfloat16)

        p2 = []
        for s in range(1, N_DEV):
            dst = lax.rem(my + s, N_DEV)
            rdma = pltpu.make_async_remote_copy(
                src_ref=g_buf.at[my],
                dst_ref=g_buf.at[my],
                send_sem=send2.at[s],
                recv_sem=recv2.at[my],
                device_id=(dst,),
                device_id_type=pl.DeviceIdType.MESH,
            )
            rdma.start()
            p2.append(rdma)

        for s in range(1, N_DEV):
            src = lax.rem(my - s + N_DEV, N_DEV)
            pltpu.make_async_remote_copy(
                src_ref=g_buf.at[src],
                dst_ref=g_buf.at[src],
                send_sem=send2.at[s],
                recv_sem=recv2.at[src],
                device_id=(src,),
                device_id_type=pl.DeviceIdType.MESH,
            ).wait_recv()

        out_ref[...] = g_buf[...].astype(jnp.float32).reshape(m, n)

        for rdma in p1 + p2:
            rdma.wait_send()

    return pl.pallas_call(
        body,
        out_shape=jax.ShapeDtypeStruct((m, n), jnp.float32),
        in_specs=[
            pl.BlockSpec(memory_space=pltpu.VMEM),
            pl.BlockSpec(memory_space=pltpu.VMEM),
        ],
        out_specs=pl.BlockSpec(memory_space=pltpu.VMEM),
        scratch_shapes=[
            pltpu.VMEM((N_DEV, chunk, n), jnp.bfloat16),
            pltpu.VMEM((N_DEV, chunk, n), jnp.bfloat16),
            pltpu.VMEM((N_DEV, chunk, n), jnp.bfloat16),
            pltpu.SemaphoreType.DMA((N_DEV,)),
            pltpu.SemaphoreType.DMA((N_DEV,)),
            pltpu.SemaphoreType.DMA((N_DEV,)),
            pltpu.SemaphoreType.DMA((N_DEV,)),
        ],
        compiler_params=pltpu.CompilerParams(collective_id=0),
    )(dy, W)


# device time: 29799 ns/iter; 1.0563x vs baseline; 1.0563x over previous
import jax
import jax.numpy as jnp
from jax import lax
from jax.experimental import pallas as pl
from jax.experimental.pallas import tpu as pltpu

N_DEV = 32


def kernel(dy, W):
    m, k = dy.shape
    n = W.shape[0]
    chunk = m // N_DEV

    def body(dy_ref, w_ref, out_ref, pbuf, rs_buf, g_buf,
             send1, recv1, send2, recv2):
        my = lax.axis_index("i")

        barrier = pltpu.get_barrier_semaphore()
        for s in range(1, N_DEV):
            peer = lax.rem(my + s, N_DEV)
            pl.semaphore_signal(
                barrier, inc=1, device_id=(peer,),
                device_id_type=pl.DeviceIdType.MESH,
            )

        partial = lax.dot_general(
            dy_ref[...].astype(jnp.bfloat16),
            w_ref[...].astype(jnp.bfloat16),
            (((1,), (1,)), ((), ())),
            preferred_element_type=jnp.float32,
        )
        pbuf[...] = partial.astype(jnp.bfloat16).reshape(N_DEV, chunk, n)

        pl.semaphore_wait(barrier, N_DEV - 1)

        p1 = []
        for s in range(1, N_DEV):
            dst = lax.rem(my + s, N_DEV)
            rdma = pltpu.make_async_remote_copy(
                src_ref=pbuf.at[dst],
                dst_ref=rs_buf.at[my],
                send_sem=send1.at[s],
                recv_sem=recv1.at[my],
                device_id=(dst,),
                device_id_type=pl.DeviceIdType.MESH,
            )
            rdma.start()
            p1.append(rdma)

        rs_buf[my, :, :] = pbuf[my]

        acc = rs_buf[my].astype(jnp.float32)
        for g in range(0, N_DEV - 1, 8):
            srcs = []
            for s in range(g + 1, min(g + 9, N_DEV)):
                src = lax.rem(my - s + N_DEV, N_DEV)
                srcs.append(src)
                pltpu.make_async_remote_copy(
                    src_ref=pbuf.at[src],
                    dst_ref=rs_buf.at[src],
                    send_sem=send1.at[s],
                    recv_sem=recv1.at[src],
                    device_id=(src,),
                    device_id_type=pl.DeviceIdType.MESH,
                ).wait_recv()
            for src in srcs:
                acc = acc + rs_buf[src].astype(jnp.float32)

        g_buf[my, :, :] = acc.astype(jnp.bfloat16)

        p2 = []
        for s in range(1, N_DEV):
            dst = lax.rem(my + s, N_DEV)
            rdma = pltpu.make_async_remote_copy(
                src_ref=g_buf.at[my],
                dst_ref=g_buf.at[my],
                send_sem=send2.at[s],
                recv_sem=recv2.at[my],
                device_id=(dst,),
                device_id_type=pl.DeviceIdType.MESH,
            )
            rdma.start()
            p2.append(rdma)

        out_ref[pl.ds(my * chunk, chunk), :] = acc

        for s in range(1, N_DEV):
            src = lax.rem(my - s + N_DEV, N_DEV)
            pltpu.make_async_remote_copy(
                src_ref=g_buf.at[src],
                dst_ref=g_buf.at[src],
                send_sem=send2.at[s],
                recv_sem=recv2.at[src],
                device_id=(src,),
                device_id_type=pl.DeviceIdType.MESH,
            ).wait_recv()
            out_ref[pl.ds(src * chunk, chunk), :] = g_buf[src].astype(
                jnp.float32
            )

        for rdma in p1 + p2:
            rdma.wait_send()

    return pl.pallas_call(
        body,
        out_shape=jax.ShapeDtypeStruct((m, n), jnp.float32),
        in_specs=[
            pl.BlockSpec(memory_space=pltpu.VMEM),
            pl.BlockSpec(memory_space=pltpu.VMEM),
        ],
        out_specs=pl.BlockSpec(memory_space=pltpu.VMEM),
        scratch_shapes=[
            pltpu.VMEM((N_DEV, chunk, n), jnp.bfloat16),
            pltpu.VMEM((N_DEV, chunk, n), jnp.bfloat16),
            pltpu.VMEM((N_DEV, chunk, n), jnp.bfloat16),
            pltpu.SemaphoreType.DMA((N_DEV,)),
            pltpu.SemaphoreType.DMA((N_DEV,)),
            pltpu.SemaphoreType.DMA((N_DEV,)),
            pltpu.SemaphoreType.DMA((N_DEV,)),
        ],
        compiler_params=pltpu.CompilerParams(collective_id=0),
    )(dy, W)
